# Initial kernel scaffold; baseline (speedup 1.0000x reference)
import jax
import jax.numpy as jnp
from jax.experimental import pallas as pl


def kernel(x, Wf, bf, Wt, bt, Wg, bg, alpha):
    raise NotImplementedError("write your pallas kernel here")


# speedup vs baseline: 1.2975x; 1.2975x over previous
import functools

import jax
import jax.numpy as jnp
from jax.experimental import pallas as pl
from jax.experimental.pallas import tpu as pltpu

F = 18
FP = 64
BT = 1024


def _gate_kernel(alpha_ref, x_ref, w_ref, b_ref, o_ref):
    logits = jnp.dot(x_ref[...], w_ref[...],
                     preferred_element_type=jnp.float32) + b_ref[...]
    f = logits[:, 0:F]
    t = logits[:, F:2 * F]
    g = logits[:, 2 * F:3 * F]

    mf = jnp.max(f, axis=-1, keepdims=True)
    ef = jnp.exp(f - mf)
    soft = ef / jnp.sum(ef, axis=-1, keepdims=True)

    idx = jax.lax.broadcasted_iota(jnp.int32, t.shape, 1)
    m1 = jnp.max(t, axis=-1, keepdims=True)
    i1 = jnp.min(jnp.where(t == m1, idx, F + 1), axis=-1, keepdims=True)
    oh1 = idx == i1
    t2 = jnp.where(oh1, -jnp.inf, t)
    m2 = jnp.max(t2, axis=-1, keepdims=True)
    i2 = jnp.min(jnp.where(t2 == m2, idx, F + 1), axis=-1, keepdims=True)
    oh2 = idx == i2
    r = jnp.exp(m2 - m1)
    v1 = 1.0 / (1.0 + r)
    v2 = r / (1.0 + r)
    s_types = jnp.where(oh1, v1, 0.0) + jnp.where(oh2, v2, 0.0)

    gates = jax.nn.sigmoid(g)
    a = jax.nn.sigmoid(alpha_ref[0])
    feats = a * s_types + (1.0 - a) * soft
    o_ref[...] = jnp.sum(gates * feats, axis=-1, keepdims=True)


@jax.jit
def kernel(x, Wf, bf, Wt, bt, Wg, bg, alpha):
    B, S, D = x.shape
    T = B * S
    x2 = x.reshape(T, D)
    Wc = jnp.concatenate([Wf, Wt, Wg], axis=0).T
    Wc = jnp.pad(Wc, ((0, 0), (0, FP - 3 * F)))
    bc = jnp.pad(jnp.concatenate([bf, bt, bg]), (0, FP - 3 * F))
    bc = bc.reshape(1, FP)

    out = pl.pallas_call(
        _gate_kernel,
        grid=(T // BT,),
        in_specs=[
            pl.BlockSpec(memory_space=pltpu.SMEM),
            pl.BlockSpec((BT, D), lambda i: (i, 0)),
            pl.BlockSpec((D, FP), lambda i: (0, 0)),
            pl.BlockSpec((1, FP), lambda i: (0, 0)),
        ],
        out_specs=pl.BlockSpec((BT, 1), lambda i: (i, 0)),
        out_shape=jax.ShapeDtypeStruct((T, 1), jnp.float32),
    )(alpha, x2, Wc, bc)
    return out.reshape(B, S, 1)


# speedup vs baseline: 2.5561x; 1.9700x over previous
import jax
import jax.numpy as jnp
from jax.experimental import pallas as pl
from jax.experimental.pallas import tpu as pltpu

F = 18
FP = 128
G = 32
BT = 1024


def _gate_kernel(alpha_ref, x_ref, w_ref, b_ref, o_ref):
    logits = jnp.dot(x_ref[...], w_ref[...],
                     preferred_element_type=jnp.float32)
    lt = logits.T + b_ref[...]
    f = lt[0:F, :]
    t = lt[G:G + F, :]
    g = lt[2 * G:2 * G + F, :]

    mf = jnp.max(f, axis=0, keepdims=True)
    ef = jnp.exp(f - mf)
    soft = ef / jnp.sum(ef, axis=0, keepdims=True)

    idx = jax.lax.broadcasted_iota(jnp.int32, t.shape, 0)
    m1 = jnp.max(t, axis=0, keepdims=True)
    i1 = jnp.min(jnp.where(t == m1, idx, F + 1), axis=0, keepdims=True)
    oh1 = idx == i1
    t2 = jnp.where(oh1, -jnp.inf, t)
    m2 = jnp.max(t2, axis=0, keepdims=True)
    i2 = jnp.min(jnp.where(t2 == m2, idx, F + 1), axis=0, keepdims=True)
    oh2 = idx == i2
    r = jnp.exp(m2 - m1)
    v1 = 1.0 / (1.0 + r)
    v2 = r / (1.0 + r)
    s_types = jnp.where(oh1, v1, 0.0) + jnp.where(oh2, v2, 0.0)

    gates = jax.nn.sigmoid(g)
    a = jax.nn.sigmoid(alpha_ref[0])
    feats = a * s_types + (1.0 - a) * soft
    o_ref[...] = jnp.sum(gates * feats, axis=0, keepdims=True)[None]


@jax.jit
def kernel(x, Wf, bf, Wt, bt, Wg, bg, alpha):
    B, S, D = x.shape
    T = B * S
    x2 = x.reshape(T, D)
    Wc = jnp.zeros((FP, D), jnp.float32)
    Wc = Wc.at[0:F].set(Wf).at[G:G + F].set(Wt).at[2 * G:2 * G + F].set(Wg)
    Wc = Wc.T
    bc = jnp.zeros((FP,), jnp.float32)
    bc = bc.at[0:F].set(bf).at[G:G + F].set(bt).at[2 * G:2 * G + F].set(bg)
    bc = bc.reshape(FP, 1)

    out = pl.pallas_call(
        _gate_kernel,
        grid=(T // BT,),
        in_specs=[
            pl.BlockSpec(memory_space=pltpu.SMEM),
            pl.BlockSpec((BT, D), lambda i: (i, 0)),
            pl.BlockSpec((D, FP), lambda i: (0, 0)),
            pl.BlockSpec((FP, 1), lambda i: (0, 0)),
        ],
        out_specs=pl.BlockSpec((1, 1, BT), lambda i: (i, 0, 0)),
        out_shape=jax.ShapeDtypeStruct((T // BT, 1, BT), jnp.float32),
    )(alpha, x2, Wc, bc)
    return out.reshape(B, S, 1)
